# Initial kernel scaffold; baseline (speedup 1.0000x reference)
import jax
import jax.numpy as jnp
from jax.experimental import pallas as pl


def kernel(source_memory, target_memory, label_memory, injection_source_logits, injection_target_logits, accumulator_source_logits, accumulator_target_logits, source, target, label, injection_source_logit, injection_target_logit, accumulator_source_logit, accumulator_target_logit):
    raise NotImplementedError("write your pallas kernel here")


# speedup vs baseline: 2.4826x; 2.4826x over previous
import functools

import jax
import jax.numpy as jnp
from jax import lax
from jax.experimental import pallas as pl
from jax.experimental.pallas import tpu as pltpu

MEMN = 500
PAD = 512
LD = 20
DW = 256
FLAT = 3 * 224 * 224
NEG = -3.0e38
POS = 3.0e38


def _small_body(a_ref, at_ref, b_ref, din_ref, nrow_ref,
                dout_ref, sidx_ref, cond_ref):
    a = a_ref[...]
    b = b_ref[...]
    key_col = jnp.max(a, axis=1, keepdims=True)
    key_row = jnp.max(at_ref[...], axis=0, keepdims=True)
    ii = lax.broadcasted_iota(jnp.int32, (PAD, PAD), 0)
    jj = lax.broadcasted_iota(jnp.int32, (PAD, PAD), 1)
    before = (key_row < key_col) | ((key_row == key_col) & (jj < ii))
    rank_col = jnp.sum(before.astype(jnp.int32), axis=1, keepdims=True)
    beforeT = (key_col < key_row) | ((key_col == key_row) & (ii < jj))
    rank_row = jnp.sum(beforeT.astype(jnp.int32), axis=0, keepdims=True)

    riota_col = lax.broadcasted_iota(jnp.int32, (PAD, 1), 0)
    onehot = rank_row == riota_col
    sidx_col = jnp.sum(jnp.where(onehot, jj, 0), axis=1, keepdims=True)

    dout = jnp.dot(onehot.astype(jnp.float32), din_ref[...],
                   preferred_element_type=jnp.float32,
                   precision=lax.Precision.HIGHEST)

    a_last = jnp.sum(jnp.where(riota_col == MEMN - 1, sidx_col, 0))
    b_last = jnp.sum(jnp.where(riota_col == a_last, sidx_col, 0))
    rows2 = lax.broadcasted_iota(jnp.int32, (PAD, 32), 0)
    thr_s = jnp.max(jnp.where(rows2 == b_last, a, NEG))
    thr_t = jnp.max(jnp.where(rows2 == b_last, b, NEG))
    nrow = nrow_ref[...]
    cols1 = lax.broadcasted_iota(jnp.int32, (1, DW), 1)
    new_s = jnp.max(jnp.where(cols1 < LD, nrow, NEG))
    new_t = jnp.max(jnp.where((cols1 >= LD) & (cols1 < 2 * LD), nrow, NEG))
    cond = (new_s >= thr_s) & ((new_s > thr_s) | (new_t > thr_t))

    lastrow = (lax.broadcasted_iota(jnp.int32, (PAD, DW), 0) == MEMN - 1) & cond
    dout_ref[...] = jnp.where(lastrow, jnp.broadcast_to(nrow, (PAD, DW)), dout)
    sidx_ref[...] = sidx_col
    cond_ref[...] = jnp.full((1, 1), cond.astype(jnp.int32))


def _gather_body(sidx_ref, cond_ref, s_ref, t_ref, ns_ref, nt_ref,
                 so_ref, to_ref):
    i = pl.program_id(0)
    take_new = (i == MEMN - 1) & (cond_ref[0] == 1)
    so_ref[...] = jnp.where(take_new, ns_ref[...], s_ref[...])
    to_ref[...] = jnp.where(take_new, nt_ref[...], t_ref[...])


def kernel(source_memory, target_memory, label_memory,
           injection_source_logits, injection_target_logits,
           accumulator_source_logits, accumulator_target_logits,
           source, target, label,
           injection_source_logit, injection_target_logit,
           accumulator_source_logit, accumulator_target_logit):
    f32 = jnp.float32
    a = jnp.full((PAD, 32), NEG, f32)
    a = a.at[:MEMN, :LD].set(injection_source_logits)
    a = a.at[MEMN:, :].set(POS)
    at = jnp.full((32, PAD), NEG, f32)
    at = at.at[:LD, :MEMN].set(injection_source_logits.T)
    at = at.at[:, MEMN:].set(POS)
    b = jnp.full((PAD, 32), NEG, f32)
    b = b.at[:MEMN, :LD].set(injection_target_logits)

    din = jnp.zeros((PAD, DW), f32)
    din = din.at[:MEMN, 0:LD].set(injection_source_logits)
    din = din.at[:MEMN, LD:2 * LD].set(injection_target_logits)
    din = din.at[:MEMN, 40:140].set(accumulator_source_logits)
    din = din.at[:MEMN, 140:240].set(accumulator_target_logits)
    din = din.at[:MEMN, 240].set(label_memory.astype(f32))

    nrow = jnp.zeros((1, DW), f32)
    nrow = nrow.at[0, 0:LD].set(injection_source_logit)
    nrow = nrow.at[0, LD:2 * LD].set(injection_target_logit)
    nrow = nrow.at[0, 40:140].set(accumulator_source_logit)
    nrow = nrow.at[0, 140:240].set(accumulator_target_logit)
    nrow = nrow.at[0, 240].set(label[0].astype(f32))

    dout, sidx2, cond2 = pl.pallas_call(
        _small_body,
        out_shape=(
            jax.ShapeDtypeStruct((PAD, DW), f32),
            jax.ShapeDtypeStruct((PAD, 1), jnp.int32),
            jax.ShapeDtypeStruct((1, 1), jnp.int32),
        ),
    )(a, at, b, din, nrow)

    sidx = sidx2[:MEMN, 0]
    cond1 = cond2.reshape((1,))

    sl = FLAT // 128
    src2 = source_memory.reshape(MEMN, sl, 128)
    tgt2 = target_memory.reshape(MEMN, sl, 128)
    ns = source.reshape(1, sl, 128)
    nt = target.reshape(1, sl, 128)

    grid_spec = pltpu.PrefetchScalarGridSpec(
        num_scalar_prefetch=2,
        grid=(MEMN,),
        in_specs=[
            pl.BlockSpec((1, sl, 128), lambda i, si, co: (si[i], 0, 0)),
            pl.BlockSpec((1, sl, 128), lambda i, si, co: (si[i], 0, 0)),
            pl.BlockSpec((1, sl, 128), lambda i, si, co: (0, 0, 0)),
            pl.BlockSpec((1, sl, 128), lambda i, si, co: (0, 0, 0)),
        ],
        out_specs=[
            pl.BlockSpec((1, sl, 128), lambda i, si, co: (i, 0, 0)),
            pl.BlockSpec((1, sl, 128), lambda i, si, co: (i, 0, 0)),
        ],
    )
    so, to = pl.pallas_call(
        _gather_body,
        grid_spec=grid_spec,
        out_shape=(
            jax.ShapeDtypeStruct((MEMN, sl, 128), f32),
            jax.ShapeDtypeStruct((MEMN, sl, 128), f32),
        ),
    )(sidx, cond1, src2, tgt2, ns, nt)

    s = so.reshape(MEMN, 3, 224, 224)
    t = to.reshape(MEMN, 3, 224, 224)
    y = dout[:MEMN, 240].astype(jnp.int32)
    ils = dout[:MEMN, 0:LD]
    ilt = dout[:MEMN, LD:2 * LD]
    als = dout[:MEMN, 40:140]
    alt = dout[:MEMN, 140:240]
    return (s, t, y, ils, ilt, als, alt)
